# baseline (speedup 1.0000x reference)
import functools

import jax
import jax.numpy as jnp
from jax import lax
from jax.experimental import pallas as pl
from jax.experimental.pallas import tpu as pltpu

N = 10000
E = 320000
H = 128
G = 64

NB = 1000
EB = 2000


def _nmap(i):
    return (i, 0)


def _wmap(i):
    return (0, 0)


def _pre_node_body(h_ref, p_ref, W_he_ref, b_he_ref, W_pe_ref, b_pe_ref,
                   h0_ref, p0_ref):
    h0_ref[...] = jnp.dot(h_ref[...], W_he_ref[...],
                          preferred_element_type=jnp.float32, precision=lax.Precision.HIGHEST) + b_he_ref[...]
    p0_ref[...] = jnp.dot(p_ref[...], W_pe_ref[...],
                          preferred_element_type=jnp.float32, precision=lax.Precision.HIGHEST) + b_pe_ref[...]


def _pre_node(h, p, W_he, b_he, W_pe, b_pe):
    return pl.pallas_call(
        _pre_node_body,
        grid=(N // NB,),
        in_specs=[
            pl.BlockSpec((NB, 128), _nmap),
            pl.BlockSpec((NB, 16), _nmap),
            pl.BlockSpec((128, H), _wmap),
            pl.BlockSpec((1, H), _wmap),
            pl.BlockSpec((16, H), _wmap),
            pl.BlockSpec((1, H), _wmap),
        ],
        out_specs=[pl.BlockSpec((NB, H), _nmap), pl.BlockSpec((NB, H), _nmap)],
        out_shape=[jax.ShapeDtypeStruct((N, H), jnp.float32),
                   jax.ShapeDtypeStruct((N, H), jnp.float32)],
    )(h, p, W_he, b_he.reshape(1, H), W_pe, b_pe.reshape(1, H))


def _pre_edge_body(e_ref, W_ee_ref, b_ee_ref, Wg_e_ref, bg_ref, e0_ref, C0_ref):
    e0 = jnp.dot(e_ref[...], W_ee_ref[...],
                 preferred_element_type=jnp.float32, precision=lax.Precision.HIGHEST) + b_ee_ref[...]
    e0_ref[...] = e0
    C0_ref[...] = jnp.dot(e0, Wg_e_ref[...],
                          preferred_element_type=jnp.float32, precision=lax.Precision.HIGHEST) + bg_ref[...]


def _pre_edge(e, W_ee, b_ee, Wg_e0, bg0):
    return pl.pallas_call(
        _pre_edge_body,
        grid=(E // EB,),
        in_specs=[
            pl.BlockSpec((EB, 16), _nmap),
            pl.BlockSpec((16, H), _wmap),
            pl.BlockSpec((1, H), _wmap),
            pl.BlockSpec((H, H), _wmap),
            pl.BlockSpec((1, H), _wmap),
        ],
        out_specs=[pl.BlockSpec((EB, H), _nmap), pl.BlockSpec((EB, H), _nmap)],
        out_shape=[jax.ShapeDtypeStruct((E, H), jnp.float32),
                   jax.ShapeDtypeStruct((E, H), jnp.float32)],
    )(e, W_ee, b_ee.reshape(1, H), Wg_e0, bg0.reshape(1, H))


def _tables_body(h_ref, p_ref, WH_ref, WP_ref, bias_ref, T_ref, B_ref, R_ref):
    o = (jnp.dot(h_ref[...], WH_ref[...], preferred_element_type=jnp.float32, precision=lax.Precision.HIGHEST)
         + jnp.dot(p_ref[...], WP_ref[...], preferred_element_type=jnp.float32, precision=lax.Precision.HIGHEST)
         + bias_ref[...])
    T_ref[...] = o[:, :3 * H]
    B_ref[...] = o[:, 3 * H:4 * H]
    R_ref[...] = o[:, 4 * H:]


def _tables(h, p, WH, WP, bias):
    return pl.pallas_call(
        _tables_body,
        grid=(N // NB,),
        in_specs=[
            pl.BlockSpec((NB, H), _nmap),
            pl.BlockSpec((NB, H), _nmap),
            pl.BlockSpec((H, 5 * H), _wmap),
            pl.BlockSpec((H, 5 * H), _wmap),
            pl.BlockSpec((1, 5 * H), _wmap),
        ],
        out_specs=[pl.BlockSpec((NB, 3 * H), _nmap),
                   pl.BlockSpec((NB, H), _nmap),
                   pl.BlockSpec((NB, H), _nmap)],
        out_shape=[jax.ShapeDtypeStruct((N, 3 * H), jnp.float32),
                   jax.ShapeDtypeStruct((N, H), jnp.float32),
                   jax.ShapeDtypeStruct((N, H), jnp.float32)],
    )(h, p, WH, WP, bias)


def _edge_last_body(Gs_ref, Br_ref, C_ref, mh_ref, mp_ref):
    z = Gs_ref[:, :H] + Br_ref[...] + C_ref[...]
    eta_hat = jax.nn.sigmoid(z)
    inv = 1.0 / jnp.sum(eta_hat, axis=1, keepdims=True)
    eta = eta_hat * inv
    mh_ref[...] = eta * Gs_ref[:, H:2 * H]
    mp_ref[...] = eta * Gs_ref[:, 2 * H:]


def _edge_first_body(Gs_ref, Br_ref, C_ref, e_ref, Wg_ref, bg_ref,
                     mh_ref, mp_ref, C1_ref):
    z = Gs_ref[:, :H] + Br_ref[...] + C_ref[...]
    eta_hat = jax.nn.sigmoid(z)
    inv = 1.0 / jnp.sum(eta_hat, axis=1, keepdims=True)
    eta = eta_hat * inv
    mh_ref[...] = eta * Gs_ref[:, H:2 * H]
    mp_ref[...] = eta * Gs_ref[:, 2 * H:]
    e1 = e_ref[...] + jnp.maximum(eta_hat, 0.0)
    C1_ref[...] = jnp.dot(e1, Wg_ref[...],
                          preferred_element_type=jnp.float32, precision=lax.Precision.HIGHEST) + bg_ref[...]


def _edge_stage(Gs, Brr, C, e_cur, Wg_e_next, bg_next, last):
    if last:
        return pl.pallas_call(
            _edge_last_body,
            grid=(E // EB,),
            in_specs=[pl.BlockSpec((EB, 3 * H), _nmap),
                      pl.BlockSpec((EB, H), _nmap),
                      pl.BlockSpec((EB, H), _nmap)],
            out_specs=[pl.BlockSpec((EB, H), _nmap),
                       pl.BlockSpec((EB, H), _nmap)],
            out_shape=[jax.ShapeDtypeStruct((E, H), jnp.float32),
                       jax.ShapeDtypeStruct((E, H), jnp.float32)],
        )(Gs, Brr, C)
    return pl.pallas_call(
        _edge_first_body,
        grid=(E // EB,),
        in_specs=[pl.BlockSpec((EB, 3 * H), _nmap),
                  pl.BlockSpec((EB, H), _nmap),
                  pl.BlockSpec((EB, H), _nmap),
                  pl.BlockSpec((EB, H), _nmap),
                  pl.BlockSpec((H, H), _wmap),
                  pl.BlockSpec((1, H), _wmap)],
        out_specs=[pl.BlockSpec((EB, H), _nmap),
                   pl.BlockSpec((EB, H), _nmap),
                   pl.BlockSpec((EB, H), _nmap)],
        out_shape=[jax.ShapeDtypeStruct((E, H), jnp.float32),
                   jax.ShapeDtypeStruct((E, H), jnp.float32),
                   jax.ShapeDtypeStruct((E, H), jnp.float32)],
    )(Gs, Brr, C, e_cur, Wg_e_next, bg_next.reshape(1, H))


def _update_body(h_ref, p_ref, R_ref, aggH_ref, aggP_ref, Wpr_ref, bpr_ref,
                 h1_ref, p1_ref):
    h_new = R_ref[...] + aggH_ref[...]
    p_new = (jnp.dot(h_new, Wpr_ref[...], preferred_element_type=jnp.float32, precision=lax.Precision.HIGHEST)
             + bpr_ref[...] + aggP_ref[...])
    h1_ref[...] = h_ref[...] + jnp.maximum(h_new, 0.0)
    p1_ref[...] = p_ref[...] + jnp.tanh(p_new)


def _update(h, p, R, aggH, aggP, Wpr, bpr):
    return pl.pallas_call(
        _update_body,
        grid=(N // NB,),
        in_specs=[pl.BlockSpec((NB, H), _nmap)] * 5 + [
            pl.BlockSpec((H, H), _wmap),
            pl.BlockSpec((1, H), _wmap)],
        out_specs=[pl.BlockSpec((NB, H), _nmap), pl.BlockSpec((NB, H), _nmap)],
        out_shape=[jax.ShapeDtypeStruct((N, H), jnp.float32),
                   jax.ShapeDtypeStruct((N, H), jnp.float32)],
    )(h, p, R, aggH, aggP, Wpr, bpr.reshape(1, H))


def _pool_body(h_ref, p_ref, b_ref, Wr1_ref, br1_ref, Wr2_ref, br2_ref,
               out_ref, ha_acc, pa_acc):
    i = pl.program_id(0)

    @pl.when(i == 0)
    def _init():
        ha_acc[...] = jnp.zeros_like(ha_acc)
        pa_acc[...] = jnp.zeros_like(pa_acc)

    b = b_ref[...]
    gid = lax.broadcasted_iota(jnp.int32, (NB, G), 1)
    onehot = (b == gid).astype(jnp.float32)
    dn = (((0,), (0,)), ((), ()))
    ha_acc[...] += lax.dot_general(onehot, h_ref[...], dn,
                                   preferred_element_type=jnp.float32, precision=lax.Precision.HIGHEST)
    pa_acc[...] += lax.dot_general(onehot, p_ref[...], dn,
                                   preferred_element_type=jnp.float32, precision=lax.Precision.HIGHEST)

    @pl.when(i == pl.num_programs(0) - 1)
    def _fin():
        cat = jnp.concatenate([ha_acc[...], pa_acc[...]], axis=1)
        r = jnp.dot(cat, Wr1_ref[...],
                    preferred_element_type=jnp.float32, precision=lax.Precision.HIGHEST) + br1_ref[...]
        r = jnp.maximum(r, 0.0)
        out_ref[...] = jnp.dot(r, Wr2_ref[...],
                               preferred_element_type=jnp.float32, precision=lax.Precision.HIGHEST) + br2_ref[...]


def _pool(h, p, batch2d, Wr1, br1, Wr2, br2):
    out = pl.pallas_call(
        _pool_body,
        grid=(N // NB,),
        in_specs=[
            pl.BlockSpec((NB, H), _nmap),
            pl.BlockSpec((NB, H), _nmap),
            pl.BlockSpec((NB, 1), _nmap),
            pl.BlockSpec((2 * H, H), _wmap),
            pl.BlockSpec((1, H), _wmap),
            pl.BlockSpec((H, 1), _wmap),
            pl.BlockSpec((1, 1), _wmap),
        ],
        out_specs=pl.BlockSpec((G, 1), _wmap),
        out_shape=jax.ShapeDtypeStruct((G, 1), jnp.float32),
        scratch_shapes=[pltpu.VMEM((G, H), jnp.float32),
                        pltpu.VMEM((G, H), jnp.float32)],
    )(h, p, batch2d, Wr1, br1.reshape(1, H), Wr2, br2.reshape(1, 1))
    return out.reshape(G)


def _gather_stage(T, B, send, rec):
    return jnp.take(T, send, axis=0), jnp.take(B, rec, axis=0)


def _scatter_stage(msgH, msgP, rec):
    aggH = jax.ops.segment_sum(msgH, rec, num_segments=N)
    aggP = jax.ops.segment_sum(msgP, rec, num_segments=N)
    return aggH, aggP


def kernel(h, e, p, edge_index, batch, W_he, b_he, W_ee, b_ee, W_pe, b_pe,
           Wg, bg, Whs, bhs, Whr, bhr, Wps, bps, Wpr, bpr, Wr1, br1, Wr2, br2):
    send = edge_index[0].astype(jnp.int32)
    rec = edge_index[1].astype(jnp.int32)
    L = Wg.shape[0]
    zeros = jnp.zeros((H, H), jnp.float32)
    zb = jnp.zeros((H,), jnp.float32)

    h, p = _pre_node(h, p, W_he, b_he, W_pe, b_pe)
    e0, C = _pre_edge(e, W_ee, b_ee, Wg[0, 2 * H:], bg[0])
    e_cur = e0

    for l in range(L):
        WH = jnp.concatenate(
            [Wg[l, :H], Whs[l, :H], zeros, Wg[l, H:2 * H], Whr[l, :H]], axis=1)
        WP = jnp.concatenate(
            [zeros, Whs[l, H:], Wps[l], zeros, Whr[l, H:]], axis=1)
        bias = jnp.concatenate([zb, bhs[l], bps[l], zb, bhr[l]]).reshape(1, 5 * H)
        T, B, R = _tables(h, p, WH, WP, bias)

        Gs, Brr = _gather_stage(T, B, send, rec)
        last = (l == L - 1)
        if last:
            msgH, msgP = _edge_stage(Gs, Brr, C, None, None, None, True)
        else:
            msgH, msgP, C = _edge_stage(Gs, Brr, C, e_cur, Wg[l + 1, 2 * H:],
                                        bg[l + 1], False)
        aggH, aggP = _scatter_stage(msgH, msgP, rec)
        h, p = _update(h, p, R, aggH, aggP, Wpr[l], bpr[l])

    return _pool(h, p, batch.astype(jnp.int32).reshape(N, 1),
                 Wr1, br1, Wr2, br2)
